# baseline (speedup 1.0000x reference)
import jax
import jax.numpy as jnp
from jax import lax
from jax.experimental import pallas as pl
from jax.experimental.pallas import tpu as pltpu
from jax.experimental.pallas import tpu_sc as plsc

HID = 256
N_MESH = 10000
N_GRID = 10000
N_EDGE = 160000

NC = 2
NS = 16
NW = NC * NS

PADE = 163840
CH = 80
GNCH = (PADE // NW) // CH
SNCH = (PADE // NS) // CH
GPAD = 10240
ROWS_PER_TILE = GPAD // NS

_SC_MESH = plsc.VectorSubcoreMesh(core_axis_name="c", subcore_axis_name="s")


def _proj_body(mesh_ref, grid_ref, wme_ref, wge_ref, wmn_ref,
               me_ref, gp_ref, mn0_ref, mn1_ref):
    m = mesh_ref[...]
    g = grid_ref[...]
    me_ref[...] = jnp.dot(
        m, wme_ref[...], preferred_element_type=jnp.float32).astype(jnp.bfloat16)
    gp_ref[...] = jnp.dot(
        g, wge_ref[...], preferred_element_type=jnp.float32).astype(jnp.bfloat16)
    mn = jnp.dot(m, wmn_ref[...], preferred_element_type=jnp.float32)
    mn0_ref[...] = mn[:, :128]
    mn1_ref[...] = mn[:, 128:]


def _proj_call(mesh, grid, wme, wge, wmn):
    blk = 2000
    n_blk = N_MESH // blk
    row_spec = pl.BlockSpec((blk, HID), lambda i: (i, 0))
    half_spec = pl.BlockSpec((blk, 128), lambda i: (i, 0))
    w_spec = pl.BlockSpec((HID, HID), lambda i: (0, 0))
    return pl.pallas_call(
        _proj_body,
        grid=(n_blk,),
        in_specs=[row_spec, row_spec, w_spec, w_spec, w_spec],
        out_specs=[row_spec, row_spec, half_spec, half_spec],
        out_shape=[
            jax.ShapeDtypeStruct((N_MESH, HID), jnp.bfloat16),
            jax.ShapeDtypeStruct((N_GRID, HID), jnp.bfloat16),
            jax.ShapeDtypeStruct((N_MESH, 128), jnp.float32),
            jax.ShapeDtypeStruct((N_MESH, 128), jnp.float32),
        ],
    )(mesh, grid, wme, wge, wmn)


def _gather_body(me_hbm, gp_hbm, src_hbm, dst_hbm, outa_hbm, outb_hbm,
                 sidx, didx, ba0, ba1, bb0, bb1,
                 sa0, sa1, sb0, sb1, swa0, swa1, swb0, swb1):
    c = lax.axis_index("c")
    s = lax.axis_index("s")
    w = s * NC + c
    pltpu.sync_copy(src_hbm.at[w], sidx)
    pltpu.sync_copy(dst_hbm.at[w], didx)
    base = w * (PADE // NW)
    bufa = (ba0, ba1)
    bufb = (bb0, bb1)
    sema = (sa0, sa1)
    semb = (sb0, sb1)
    semwa = (swa0, swa1)
    semwb = (swb0, swb1)

    def issue(cw, slot):
        pltpu.async_copy(me_hbm.at[sidx.at[cw]], bufa[slot], sema[slot])
        pltpu.async_copy(gp_hbm.at[didx.at[cw]], bufb[slot], semb[slot])

    def outa_slice(cw):
        return outa_hbm.at[pl.ds(base + cw * CH, CH)]

    def outb_slice(cw):
        return outb_hbm.at[pl.ds(base + cw * CH, CH)]

    issue(0, 0)

    def pair(g, carry):
        for b in (0, 1):
            cidx = 2 * g + b
            nxt = cidx + 1

            def prefetch():
                def drain_wb():
                    pltpu.make_async_copy(
                        bufa[1 - b], outa_slice(cidx - 1), semwa[1 - b]).wait()
                    pltpu.make_async_copy(
                        bufb[1 - b], outb_slice(cidx - 1), semwb[1 - b]).wait()
                pl.when(cidx >= 1)(drain_wb)
                issue(nxt, 1 - b)

            pl.when(nxt < GNCH)(prefetch)
            pltpu.make_async_copy(
                me_hbm.at[sidx.at[cidx]], bufa[b], sema[b]).wait()
            pltpu.make_async_copy(
                gp_hbm.at[didx.at[cidx]], bufb[b], semb[b]).wait()
            pltpu.async_copy(bufa[b], outa_slice(cidx), semwa[b])
            pltpu.async_copy(bufb[b], outb_slice(cidx), semwb[b])
        return carry

    lax.fori_loop(0, GNCH // 2, pair, 0)
    pltpu.make_async_copy(bufa[0], outa_slice(GNCH - 2), semwa[0]).wait()
    pltpu.make_async_copy(bufa[1], outa_slice(GNCH - 1), semwa[1]).wait()
    pltpu.make_async_copy(bufb[0], outb_slice(GNCH - 2), semwb[0]).wait()
    pltpu.make_async_copy(bufb[1], outb_slice(GNCH - 1), semwb[1]).wait()


def _gather_call(me_proj, gp_proj, src3, dst3):
    out = jax.ShapeDtypeStruct((PADE, HID // 2), jnp.int32)
    fn = pl.kernel(
        _gather_body,
        out_type=(out, out),
        mesh=_SC_MESH,
        scratch_types=[
            pltpu.VMEM((GNCH, CH), jnp.int32),
            pltpu.VMEM((GNCH, CH), jnp.int32),
            pltpu.VMEM((CH, HID // 2), jnp.int32),
            pltpu.VMEM((CH, HID // 2), jnp.int32),
            pltpu.VMEM((CH, HID // 2), jnp.int32),
            pltpu.VMEM((CH, HID // 2), jnp.int32),
            pltpu.SemaphoreType.DMA,
            pltpu.SemaphoreType.DMA,
            pltpu.SemaphoreType.DMA,
            pltpu.SemaphoreType.DMA,
            pltpu.SemaphoreType.DMA,
            pltpu.SemaphoreType.DMA,
            pltpu.SemaphoreType.DMA,
            pltpu.SemaphoreType.DMA,
        ],
    )
    return fn(me_proj, gp_proj, src3, dst3)


def _edge_body(e_ref, ga_ref, gb_ref, w1_ref, b1_ref, w2_ref, b2_ref,
               gln_ref, bln_ref, o0_ref, o1_ref):
    x = e_ref[...]
    g = ga_ref[...].astype(jnp.float32) + gb_ref[...].astype(jnp.float32)
    h = jnp.dot(x, w1_ref[...], preferred_element_type=jnp.float32)
    h = h + g + b1_ref[...]
    h = h * jax.nn.sigmoid(h)
    h = jnp.dot(h, w2_ref[...], preferred_element_type=jnp.float32) + b2_ref[...]
    mu = jnp.mean(h, axis=-1, keepdims=True)
    var = jnp.mean((h - mu) * (h - mu), axis=-1, keepdims=True)
    h = (h - mu) * lax.rsqrt(var + 1e-5) * gln_ref[...] + bln_ref[...]
    out = h + x
    o0_ref[...] = out[:, :128]
    o1_ref[...] = out[:, 128:]


def _edge_call(e, ga, gb, w1, b1, w2, b2, gln, bln):
    blk = 2000
    n_blk = N_EDGE // blk
    row_spec = pl.BlockSpec((blk, HID), lambda i: (i, 0))
    half_spec = pl.BlockSpec((blk, 128), lambda i: (i, 0))
    w_spec = pl.BlockSpec((HID, HID), lambda i: (0, 0))
    b_spec = pl.BlockSpec((1, HID), lambda i: (0, 0))
    return pl.pallas_call(
        _edge_body,
        grid=(n_blk,),
        in_specs=[row_spec, row_spec, row_spec, w_spec, b_spec, w_spec,
                  b_spec, b_spec, b_spec],
        out_specs=[half_spec, half_spec],
        out_shape=[
            jax.ShapeDtypeStruct((PADE, 128), jnp.float32),
            jax.ShapeDtypeStruct((PADE, 128), jnp.float32),
        ],
    )(e, ga, gb, w1, b1, w2, b2, gln, bln)


def _agg_e_body(ef0, ef1, dst3, zeros_h,
                ae0, ae1, spmem, didxb, db0, db1, sd0, sd1, si0, si1):
    c = lax.axis_index("c")
    t = lax.axis_index("s")
    rows = pl.ds(t * ROWS_PER_TILE, ROWS_PER_TILE)
    ebase = t * (PADE // NS)
    dbuf = (db0, db1)
    semd = (sd0, sd1)
    semi = (si0, si1)

    def load_didx(cw, slot):
        pltpu.async_copy(dst3.at[t, cw], didxb.at[slot], semi[slot])

    def wait_didx(cw, slot):
        pltpu.make_async_copy(
            dst3.at[t, cw], didxb.at[slot], semi[slot]).wait()

    def run(efc, aec):
        pltpu.sync_copy(zeros_h.at[rows], spmem.at[rows])
        plsc.subcore_barrier()

        def load_a(cw, slot):
            pltpu.async_copy(
                efc.at[pl.ds(ebase + cw * CH, CH)], dbuf[slot], semd[slot])

        load_a(0, 0)
        load_didx(0, 0)

        def pair_a(g, carry):
            for b in (0, 1):
                cidx = 2 * g + b

                def prefetch():
                    load_a(cidx + 1, 1 - b)
                    load_didx(cidx + 1, 1 - b)

                pl.when(cidx + 1 < SNCH)(prefetch)
                pltpu.make_async_copy(
                    efc.at[pl.ds(ebase + cidx * CH, CH)],
                    dbuf[b], semd[b]).wait()
                wait_didx(cidx, b)
                pltpu.sync_copy(dbuf[b], spmem.at[didxb.at[b]], add=True)
            return carry

        lax.fori_loop(0, SNCH // 2, pair_a, 0)
        plsc.subcore_barrier()
        pltpu.sync_copy(spmem.at[rows], aec.at[rows])

    pl.when(c == 0)(lambda: run(ef0, ae0))
    pl.when(c == 1)(lambda: run(ef1, ae1))


def _agg_e_call(ef0, ef1, dst3, zeros_h):
    out = jax.ShapeDtypeStruct((GPAD, 128), jnp.float32)
    fn = pl.kernel(
        _agg_e_body,
        out_type=(out, out),
        mesh=_SC_MESH,
        scratch_types=[
            pltpu.VMEM_SHARED((GPAD, 128), jnp.float32),
            pltpu.VMEM((2, CH), jnp.int32),
            pltpu.VMEM((CH, 128), jnp.float32),
            pltpu.VMEM((CH, 128), jnp.float32),
            pltpu.SemaphoreType.DMA,
            pltpu.SemaphoreType.DMA,
            pltpu.SemaphoreType.DMA,
            pltpu.SemaphoreType.DMA,
        ],
    )
    return fn(ef0, ef1, dst3, zeros_h)


def _agg_s_body(mn0, mn1, dst3, src3, zeros_h,
                as0, as1, spmem, didxb, sidx, db0, db1, sd0, sd1, si0, si1):
    c = lax.axis_index("c")
    t = lax.axis_index("s")
    rows = pl.ds(t * ROWS_PER_TILE, ROWS_PER_TILE)
    dbuf = (db0, db1)
    semd = (sd0, sd1)
    semi = (si0, si1)

    def load_didx(cw, slot):
        pltpu.async_copy(dst3.at[t, cw], didxb.at[slot], semi[slot])

    def wait_didx(cw, slot):
        pltpu.make_async_copy(
            dst3.at[t, cw], didxb.at[slot], semi[slot]).wait()

    def run(mnc, asc):
        pltpu.sync_copy(src3.at[t], sidx)
        pltpu.sync_copy(zeros_h.at[rows], spmem.at[rows])
        plsc.subcore_barrier()

        def load_b(cw, slot):
            pltpu.async_copy(mnc.at[sidx.at[cw]], dbuf[slot], semd[slot])

        load_b(0, 0)
        load_didx(0, 0)

        def pair_b(g, carry):
            for b in (0, 1):
                cidx = 2 * g + b

                def prefetch():
                    load_b(cidx + 1, 1 - b)
                    load_didx(cidx + 1, 1 - b)

                pl.when(cidx + 1 < SNCH)(prefetch)
                pltpu.make_async_copy(
                    mnc.at[sidx.at[cidx]], dbuf[b], semd[b]).wait()
                wait_didx(cidx, b)
                pltpu.sync_copy(dbuf[b], spmem.at[didxb.at[b]], add=True)
            return carry

        lax.fori_loop(0, SNCH // 2, pair_b, 0)
        plsc.subcore_barrier()
        pltpu.sync_copy(spmem.at[rows], asc.at[rows])

    pl.when(c == 0)(lambda: run(mn0, as0))
    pl.when(c == 1)(lambda: run(mn1, as1))


def _agg_s_call(mn0, mn1, dst3, src3, zeros_h):
    out = jax.ShapeDtypeStruct((GPAD, 128), jnp.float32)
    fn = pl.kernel(
        _agg_s_body,
        out_type=(out, out),
        mesh=_SC_MESH,
        scratch_types=[
            pltpu.VMEM_SHARED((GPAD, 128), jnp.float32),
            pltpu.VMEM((2, CH), jnp.int32),
            pltpu.VMEM((SNCH, CH), jnp.int32),
            pltpu.VMEM((CH, 128), jnp.float32),
            pltpu.VMEM((CH, 128), jnp.float32),
            pltpu.SemaphoreType.DMA,
            pltpu.SemaphoreType.DMA,
            pltpu.SemaphoreType.DMA,
            pltpu.SemaphoreType.DMA,
        ],
    )
    return fn(mn0, mn1, dst3, src3, zeros_h)


def _node_body(ae0_ref, ae1_ref, as0_ref, as1_ref, grid_ref,
               wa_ref, wb_ref, b1_ref, w2_ref, b2_ref, gln_ref, bln_ref,
               out_ref):
    pre = jnp.concatenate([as0_ref[...], as1_ref[...]], axis=-1)
    pre = pre + jnp.dot(ae0_ref[...], wa_ref[...],
                        preferred_element_type=jnp.float32)
    pre = pre + jnp.dot(ae1_ref[...], wb_ref[...],
                        preferred_element_type=jnp.float32)
    pre = pre + b1_ref[...]
    h = pre * jax.nn.sigmoid(pre)
    h = jnp.dot(h, w2_ref[...], preferred_element_type=jnp.float32) + b2_ref[...]
    mu = jnp.mean(h, axis=-1, keepdims=True)
    var = jnp.mean((h - mu) * (h - mu), axis=-1, keepdims=True)
    h = (h - mu) * lax.rsqrt(var + 1e-5) * gln_ref[...] + bln_ref[...]
    out_ref[...] = h + grid_ref[...]


def _node_call(ae0, ae1, as0, as1, grid, wa, wb, b1, w2, b2, gln, bln):
    blk = 2000
    n_blk = N_GRID // blk
    row_spec = pl.BlockSpec((blk, HID), lambda i: (i, 0))
    half_spec = pl.BlockSpec((blk, 128), lambda i: (i, 0))
    wh_spec = pl.BlockSpec((128, HID), lambda i: (0, 0))
    w_spec = pl.BlockSpec((HID, HID), lambda i: (0, 0))
    b_spec = pl.BlockSpec((1, HID), lambda i: (0, 0))
    return pl.pallas_call(
        _node_body,
        grid=(n_blk,),
        in_specs=[half_spec, half_spec, half_spec, half_spec, row_spec,
                  wh_spec, wh_spec, b_spec, w_spec, b_spec, b_spec, b_spec],
        out_specs=row_spec,
        out_shape=jax.ShapeDtypeStruct((N_GRID, HID), jnp.float32),
    )(ae0, ae1, as0, as1, grid, wa, wb, b1, w2, b2, gln, bln)


def kernel(mesh2grid_edge_features, grid_node_features, mesh_node_features,
           mesh2graph_edge_indices_src, mesh2graph_edge_indices_dst,
           eW1, eb1, eW2, eb2, eg, ebt,
           nW1, nb1, nW2, nb2, ng, nbt):
    src = mesh2graph_edge_indices_src.astype(jnp.int32)
    dst = mesh2graph_edge_indices_dst.astype(jnp.int32)
    npad = PADE - N_EDGE
    pad_iota = jnp.arange(npad, dtype=jnp.int32)
    src_p = jnp.concatenate([src, pad_iota % N_MESH])
    dst_p = jnp.concatenate([dst, pad_iota % N_GRID])
    dst_t = jnp.concatenate([dst, N_GRID + pad_iota % (GPAD - N_GRID)])

    me_proj, gp_proj, mn0, mn1 = _proj_call(
        mesh_node_features, grid_node_features,
        eW1[256:512], eW1[512:768], nW1[:256])

    def as_i32(x):
        return lax.bitcast_convert_type(
            x.reshape(x.shape[0], x.shape[1] // 2, 2), jnp.int32)

    ga_i32, gb_i32 = _gather_call(
        as_i32(me_proj), as_i32(gp_proj),
        src_p.reshape(NW, GNCH, CH), dst_p.reshape(NW, GNCH, CH))

    def as_bf16(x):
        return lax.bitcast_convert_type(x, jnp.bfloat16).reshape(PADE, HID)

    zeros_h = jnp.zeros((GPAD, 128), jnp.float32)
    dst_sc = dst_t.reshape(NS, SNCH, CH)
    as0, as1 = _agg_s_call(
        mn0, mn1, dst_sc, src_p.reshape(NS, SNCH, CH), zeros_h)

    ef0, ef1 = _edge_call(
        mesh2grid_edge_features, as_bf16(ga_i32), as_bf16(gb_i32),
        eW1[:256], eb1.reshape(1, HID), eW2, eb2.reshape(1, HID),
        eg.reshape(1, HID), ebt.reshape(1, HID))

    ae0, ae1 = _agg_e_call(ef0, ef1, dst_sc, zeros_h)

    out = _node_call(
        ae0, ae1, as0, as1,
        grid_node_features,
        nW1[256:384], nW1[384:512], nb1.reshape(1, HID),
        nW2, nb2.reshape(1, HID), ng.reshape(1, HID), nbt.reshape(1, HID))
    return out


# speedup vs baseline: 3.9541x; 3.9541x over previous
import jax
import jax.numpy as jnp
from jax import lax
from jax.experimental import pallas as pl
from jax.experimental.pallas import tpu as pltpu
from jax.experimental.pallas import tpu_sc as plsc

HID = 256
N_MESH = 10000
N_GRID = 10000
N_EDGE = 160000

NC = 2
NS = 16
NW = NC * NS

PADE = 163840
CH = 80
GNCH = (PADE // NW) // CH
SNCH = (PADE // NS) // CH
GPAD = 10240
ROWS_PER_TILE = GPAD // NS

_SC_MESH = plsc.VectorSubcoreMesh(core_axis_name="c", subcore_axis_name="s")


def _proj_body(mesh_ref, grid_ref, wme_ref, wge_ref, wmn_ref,
               me_ref, gp_ref, mn0_ref, mn1_ref):
    m = mesh_ref[...]
    g = grid_ref[...]

    def pack(x):
        bits = lax.bitcast_convert_type(x.astype(jnp.bfloat16), jnp.uint16)
        lo = bits[:, :128].astype(jnp.uint32)
        hi = bits[:, 128:].astype(jnp.uint32)
        return lax.bitcast_convert_type(lo | (hi << 16), jnp.int32)

    me_ref[...] = pack(jnp.dot(m, wme_ref[...],
                               preferred_element_type=jnp.float32))
    gp_ref[...] = pack(jnp.dot(g, wge_ref[...],
                               preferred_element_type=jnp.float32))
    mn = jnp.dot(m, wmn_ref[...], preferred_element_type=jnp.float32)
    mn0_ref[...] = mn[:, :128]
    mn1_ref[...] = mn[:, 128:]


def _proj_call(mesh, grid, wme, wge, wmn):
    blk = 2000
    n_blk = N_MESH // blk
    row_spec = pl.BlockSpec((blk, HID), lambda i: (i, 0))
    half_spec = pl.BlockSpec((blk, 128), lambda i: (i, 0))
    w_spec = pl.BlockSpec((HID, HID), lambda i: (0, 0))
    return pl.pallas_call(
        _proj_body,
        grid=(n_blk,),
        in_specs=[row_spec, row_spec, w_spec, w_spec, w_spec],
        out_specs=[half_spec, half_spec, half_spec, half_spec],
        out_shape=[
            jax.ShapeDtypeStruct((N_MESH, 128), jnp.int32),
            jax.ShapeDtypeStruct((N_GRID, 128), jnp.int32),
            jax.ShapeDtypeStruct((N_MESH, 128), jnp.float32),
            jax.ShapeDtypeStruct((N_MESH, 128), jnp.float32),
        ],
    )(mesh, grid, wme, wge, wmn)


def _gather_body(me_hbm, gp_hbm, src_hbm, dst_hbm, outa_hbm, outb_hbm,
                 sidx, didx, ba0, ba1, bb0, bb1,
                 sa0, sa1, sb0, sb1, swa0, swa1, swb0, swb1):
    c = lax.axis_index("c")
    s = lax.axis_index("s")
    w = s * NC + c
    pltpu.sync_copy(src_hbm.at[w], sidx)
    pltpu.sync_copy(dst_hbm.at[w], didx)
    base = w * (PADE // NW)
    bufa = (ba0, ba1)
    bufb = (bb0, bb1)
    sema = (sa0, sa1)
    semb = (sb0, sb1)
    semwa = (swa0, swa1)
    semwb = (swb0, swb1)

    def issue(cw, slot):
        pltpu.async_copy(me_hbm.at[sidx.at[cw]], bufa[slot], sema[slot])
        pltpu.async_copy(gp_hbm.at[didx.at[cw]], bufb[slot], semb[slot])

    def outa_slice(cw):
        return outa_hbm.at[pl.ds(base + cw * CH, CH)]

    def outb_slice(cw):
        return outb_hbm.at[pl.ds(base + cw * CH, CH)]

    issue(0, 0)

    def pair(g, carry):
        for b in (0, 1):
            cidx = 2 * g + b
            nxt = cidx + 1

            def prefetch():
                def drain_wb():
                    pltpu.make_async_copy(
                        bufa[1 - b], outa_slice(cidx - 1), semwa[1 - b]).wait()
                    pltpu.make_async_copy(
                        bufb[1 - b], outb_slice(cidx - 1), semwb[1 - b]).wait()
                pl.when(cidx >= 1)(drain_wb)
                issue(nxt, 1 - b)

            pl.when(nxt < GNCH)(prefetch)
            pltpu.make_async_copy(
                me_hbm.at[sidx.at[cidx]], bufa[b], sema[b]).wait()
            pltpu.make_async_copy(
                gp_hbm.at[didx.at[cidx]], bufb[b], semb[b]).wait()
            pltpu.async_copy(bufa[b], outa_slice(cidx), semwa[b])
            pltpu.async_copy(bufb[b], outb_slice(cidx), semwb[b])
        return carry

    lax.fori_loop(0, GNCH // 2, pair, 0)
    pltpu.make_async_copy(bufa[0], outa_slice(GNCH - 2), semwa[0]).wait()
    pltpu.make_async_copy(bufa[1], outa_slice(GNCH - 1), semwa[1]).wait()
    pltpu.make_async_copy(bufb[0], outb_slice(GNCH - 2), semwb[0]).wait()
    pltpu.make_async_copy(bufb[1], outb_slice(GNCH - 1), semwb[1]).wait()


def _gather_call(me_proj, gp_proj, src3, dst3):
    out = jax.ShapeDtypeStruct((PADE, HID // 2), jnp.int32)
    fn = pl.kernel(
        _gather_body,
        out_type=(out, out),
        mesh=_SC_MESH,
        scratch_types=[
            pltpu.VMEM((GNCH, CH), jnp.int32),
            pltpu.VMEM((GNCH, CH), jnp.int32),
            pltpu.VMEM((CH, HID // 2), jnp.int32),
            pltpu.VMEM((CH, HID // 2), jnp.int32),
            pltpu.VMEM((CH, HID // 2), jnp.int32),
            pltpu.VMEM((CH, HID // 2), jnp.int32),
            pltpu.SemaphoreType.DMA,
            pltpu.SemaphoreType.DMA,
            pltpu.SemaphoreType.DMA,
            pltpu.SemaphoreType.DMA,
            pltpu.SemaphoreType.DMA,
            pltpu.SemaphoreType.DMA,
            pltpu.SemaphoreType.DMA,
            pltpu.SemaphoreType.DMA,
        ],
    )
    return fn(me_proj, gp_proj, src3, dst3)


def _edge_body(e_ref, ga_ref, gb_ref, w1_ref, b1_ref, w2_ref, b2_ref,
               gln_ref, bln_ref, o0_ref, o1_ref):
    x = e_ref[...]

    def unpack(w):
        wu = lax.bitcast_convert_type(w, jnp.uint32)
        lo = lax.bitcast_convert_type(wu << 16, jnp.float32)
        hi = lax.bitcast_convert_type(wu & jnp.uint32(0xFFFF0000), jnp.float32)
        return jnp.concatenate([lo, hi], axis=-1)

    g = unpack(ga_ref[...]) + unpack(gb_ref[...])
    h = jnp.dot(x, w1_ref[...], preferred_element_type=jnp.float32)
    h = h + g + b1_ref[...]
    h = h * jax.nn.sigmoid(h)
    h = jnp.dot(h, w2_ref[...], preferred_element_type=jnp.float32) + b2_ref[...]
    mu = jnp.mean(h, axis=-1, keepdims=True)
    var = jnp.mean((h - mu) * (h - mu), axis=-1, keepdims=True)
    h = (h - mu) * lax.rsqrt(var + 1e-5) * gln_ref[...] + bln_ref[...]
    out = h + x
    o0_ref[...] = out[:, :128]
    o1_ref[...] = out[:, 128:]


def _edge_call(e, ga, gb, w1, b1, w2, b2, gln, bln):
    blk = 2000
    n_blk = N_EDGE // blk
    row_spec = pl.BlockSpec((blk, HID), lambda i: (i, 0))
    half_spec = pl.BlockSpec((blk, 128), lambda i: (i, 0))
    w_spec = pl.BlockSpec((HID, HID), lambda i: (0, 0))
    b_spec = pl.BlockSpec((1, HID), lambda i: (0, 0))
    return pl.pallas_call(
        _edge_body,
        grid=(n_blk,),
        in_specs=[row_spec, half_spec, half_spec, w_spec, b_spec, w_spec,
                  b_spec, b_spec, b_spec],
        out_specs=[half_spec, half_spec],
        out_shape=[
            jax.ShapeDtypeStruct((PADE, 128), jnp.float32),
            jax.ShapeDtypeStruct((PADE, 128), jnp.float32),
        ],
    )(e, ga, gb, w1, b1, w2, b2, gln, bln)


def _agg_e_body(ef0, ef1, dst3, zeros_h,
                ae0, ae1, spmem, didxb, db0, db1, sd0, sd1, si0, si1):
    c = lax.axis_index("c")
    t = lax.axis_index("s")
    rows = pl.ds(t * ROWS_PER_TILE, ROWS_PER_TILE)
    ebase = t * (PADE // NS)
    dbuf = (db0, db1)
    semd = (sd0, sd1)
    semi = (si0, si1)

    def load_didx(cw, slot):
        pltpu.async_copy(dst3.at[t, cw], didxb.at[slot], semi[slot])

    def wait_didx(cw, slot):
        pltpu.make_async_copy(
            dst3.at[t, cw], didxb.at[slot], semi[slot]).wait()

    def run(efc, aec):
        pltpu.sync_copy(zeros_h.at[rows], spmem.at[rows])
        plsc.subcore_barrier()

        def load_a(cw, slot):
            pltpu.async_copy(
                efc.at[pl.ds(ebase + cw * CH, CH)], dbuf[slot], semd[slot])

        load_a(0, 0)
        load_didx(0, 0)

        def pair_a(g, carry):
            for b in (0, 1):
                cidx = 2 * g + b

                def prefetch():
                    load_a(cidx + 1, 1 - b)
                    load_didx(cidx + 1, 1 - b)

                pl.when(cidx + 1 < SNCH)(prefetch)
                pltpu.make_async_copy(
                    efc.at[pl.ds(ebase + cidx * CH, CH)],
                    dbuf[b], semd[b]).wait()
                wait_didx(cidx, b)
                pltpu.sync_copy(dbuf[b], spmem.at[didxb.at[b]], add=True)
            return carry

        lax.fori_loop(0, SNCH // 2, pair_a, 0)
        plsc.subcore_barrier()
        pltpu.sync_copy(spmem.at[rows], aec.at[rows])

    pl.when(c == 0)(lambda: run(ef0, ae0))
    pl.when(c == 1)(lambda: run(ef1, ae1))


def _agg_e_call(ef0, ef1, dst3, zeros_h):
    out = jax.ShapeDtypeStruct((GPAD, 128), jnp.float32)
    fn = pl.kernel(
        _agg_e_body,
        out_type=(out, out),
        mesh=_SC_MESH,
        scratch_types=[
            pltpu.VMEM_SHARED((GPAD, 128), jnp.float32),
            pltpu.VMEM((2, CH), jnp.int32),
            pltpu.VMEM((CH, 128), jnp.float32),
            pltpu.VMEM((CH, 128), jnp.float32),
            pltpu.SemaphoreType.DMA,
            pltpu.SemaphoreType.DMA,
            pltpu.SemaphoreType.DMA,
            pltpu.SemaphoreType.DMA,
        ],
    )
    return fn(ef0, ef1, dst3, zeros_h)


def _agg_s_body(mn0, mn1, dst3, src3, zeros_h,
                as0, as1, spmem, didxb, sidx, db0, db1, sd0, sd1, si0, si1):
    c = lax.axis_index("c")
    t = lax.axis_index("s")
    rows = pl.ds(t * ROWS_PER_TILE, ROWS_PER_TILE)
    dbuf = (db0, db1)
    semd = (sd0, sd1)
    semi = (si0, si1)

    def load_didx(cw, slot):
        pltpu.async_copy(dst3.at[t, cw], didxb.at[slot], semi[slot])

    def wait_didx(cw, slot):
        pltpu.make_async_copy(
            dst3.at[t, cw], didxb.at[slot], semi[slot]).wait()

    def run(mnc, asc):
        pltpu.sync_copy(src3.at[t], sidx)
        pltpu.sync_copy(zeros_h.at[rows], spmem.at[rows])
        plsc.subcore_barrier()

        def load_b(cw, slot):
            pltpu.async_copy(mnc.at[sidx.at[cw]], dbuf[slot], semd[slot])

        load_b(0, 0)
        load_didx(0, 0)

        def pair_b(g, carry):
            for b in (0, 1):
                cidx = 2 * g + b

                def prefetch():
                    load_b(cidx + 1, 1 - b)
                    load_didx(cidx + 1, 1 - b)

                pl.when(cidx + 1 < SNCH)(prefetch)
                pltpu.make_async_copy(
                    mnc.at[sidx.at[cidx]], dbuf[b], semd[b]).wait()
                wait_didx(cidx, b)
                pltpu.sync_copy(dbuf[b], spmem.at[didxb.at[b]], add=True)
            return carry

        lax.fori_loop(0, SNCH // 2, pair_b, 0)
        plsc.subcore_barrier()
        pltpu.sync_copy(spmem.at[rows], asc.at[rows])

    pl.when(c == 0)(lambda: run(mn0, as0))
    pl.when(c == 1)(lambda: run(mn1, as1))


def _agg_s_call(mn0, mn1, dst3, src3, zeros_h):
    out = jax.ShapeDtypeStruct((GPAD, 128), jnp.float32)
    fn = pl.kernel(
        _agg_s_body,
        out_type=(out, out),
        mesh=_SC_MESH,
        scratch_types=[
            pltpu.VMEM_SHARED((GPAD, 128), jnp.float32),
            pltpu.VMEM((2, CH), jnp.int32),
            pltpu.VMEM((SNCH, CH), jnp.int32),
            pltpu.VMEM((CH, 128), jnp.float32),
            pltpu.VMEM((CH, 128), jnp.float32),
            pltpu.SemaphoreType.DMA,
            pltpu.SemaphoreType.DMA,
            pltpu.SemaphoreType.DMA,
            pltpu.SemaphoreType.DMA,
        ],
    )
    return fn(mn0, mn1, dst3, src3, zeros_h)


def _node_body(ae0_ref, ae1_ref, as0_ref, as1_ref, grid_ref,
               wa_ref, wb_ref, b1_ref, w2_ref, b2_ref, gln_ref, bln_ref,
               out_ref):
    pre = jnp.concatenate([as0_ref[...], as1_ref[...]], axis=-1)
    pre = pre + jnp.dot(ae0_ref[...], wa_ref[...],
                        preferred_element_type=jnp.float32)
    pre = pre + jnp.dot(ae1_ref[...], wb_ref[...],
                        preferred_element_type=jnp.float32)
    pre = pre + b1_ref[...]
    h = pre * jax.nn.sigmoid(pre)
    h = jnp.dot(h, w2_ref[...], preferred_element_type=jnp.float32) + b2_ref[...]
    mu = jnp.mean(h, axis=-1, keepdims=True)
    var = jnp.mean((h - mu) * (h - mu), axis=-1, keepdims=True)
    h = (h - mu) * lax.rsqrt(var + 1e-5) * gln_ref[...] + bln_ref[...]
    out_ref[...] = h + grid_ref[...]


def _node_call(ae0, ae1, as0, as1, grid, wa, wb, b1, w2, b2, gln, bln):
    blk = 2000
    n_blk = N_GRID // blk
    row_spec = pl.BlockSpec((blk, HID), lambda i: (i, 0))
    half_spec = pl.BlockSpec((blk, 128), lambda i: (i, 0))
    wh_spec = pl.BlockSpec((128, HID), lambda i: (0, 0))
    w_spec = pl.BlockSpec((HID, HID), lambda i: (0, 0))
    b_spec = pl.BlockSpec((1, HID), lambda i: (0, 0))
    return pl.pallas_call(
        _node_body,
        grid=(n_blk,),
        in_specs=[half_spec, half_spec, half_spec, half_spec, row_spec,
                  wh_spec, wh_spec, b_spec, w_spec, b_spec, b_spec, b_spec],
        out_specs=row_spec,
        out_shape=jax.ShapeDtypeStruct((N_GRID, HID), jnp.float32),
    )(ae0, ae1, as0, as1, grid, wa, wb, b1, w2, b2, gln, bln)


def kernel(mesh2grid_edge_features, grid_node_features, mesh_node_features,
           mesh2graph_edge_indices_src, mesh2graph_edge_indices_dst,
           eW1, eb1, eW2, eb2, eg, ebt,
           nW1, nb1, nW2, nb2, ng, nbt):
    src = mesh2graph_edge_indices_src.astype(jnp.int32)
    dst = mesh2graph_edge_indices_dst.astype(jnp.int32)
    npad = PADE - N_EDGE
    pad_iota = jnp.arange(npad, dtype=jnp.int32)
    src_p = jnp.concatenate([src, pad_iota % N_MESH])
    dst_p = jnp.concatenate([dst, pad_iota % N_GRID])
    dst_t = jnp.concatenate([dst, N_GRID + pad_iota % (GPAD - N_GRID)])

    me_proj, gp_proj, mn0, mn1 = _proj_call(
        mesh_node_features, grid_node_features,
        eW1[256:512], eW1[512:768], nW1[:256])

    ga_i32, gb_i32 = _gather_call(
        me_proj, gp_proj,
        src_p.reshape(NW, GNCH, CH), dst_p.reshape(NW, GNCH, CH))

    zeros_h = jnp.zeros((GPAD, 128), jnp.float32)
    dst_sc = dst_t.reshape(NS, SNCH, CH)
    as0, as1 = _agg_s_call(
        mn0, mn1, dst_sc, src_p.reshape(NS, SNCH, CH), zeros_h)

    ef0, ef1 = _edge_call(
        mesh2grid_edge_features, ga_i32, gb_i32,
        eW1[:256], eb1.reshape(1, HID), eW2, eb2.reshape(1, HID),
        eg.reshape(1, HID), ebt.reshape(1, HID))

    ae0, ae1 = _agg_e_call(ef0, ef1, dst_sc, zeros_h)

    out = _node_call(
        ae0, ae1, as0, as1,
        grid_node_features,
        nW1[256:384], nW1[384:512], nb1.reshape(1, HID),
        nW2, nb2.reshape(1, HID), ng.reshape(1, HID), nbt.reshape(1, HID))
    return out
